# baseline (speedup 1.0000x reference)
import functools

import jax
import jax.numpy as jnp
import numpy as np
from jax.experimental import pallas as pl
from jax.experimental.pallas import tpu as pltpu

_NB = 64
_LOG2E = 1.4426950408889634
_BW = 1.0 / _NB
_DENOM = 2.0 * _BW * _BW + 1e-7
_SCALE = np.float32(np.sqrt(_LOG2E / _DENOM))
_CSTEP = np.float32(1.0 / (_NB - 1))


def _hist_body(x_ref, t_ref, ph_ref, th_ref, *, rows):
    j = pl.program_id(1)

    centers = (jax.lax.broadcasted_iota(jnp.int32, (_NB, 128), 0)
               .astype(jnp.float32) * np.float32(_CSTEP * _SCALE))

    def accum2(ref_a, ref_b):
        sa = jnp.zeros((_NB, 128), jnp.float32)
        sc = jnp.zeros((_NB, 128), jnp.float32)
        ba = jnp.zeros((_NB, 128), jnp.bfloat16)
        bb = jnp.zeros((_NB, 128), jnp.bfloat16)
        for k in range(rows):
            xa = ref_a[0, k : k + 1, :] * _SCALE
            xb = ref_b[0, k : k + 1, :] * _SCALE
            da = (xa - centers).astype(jnp.bfloat16)
            db = (xb - centers).astype(jnp.bfloat16)
            ba = ba + jnp.exp2(-(da * da))
            bb = bb + jnp.exp2(-(db * db))
            if (k + 1) % 16 == 0:
                sa = sa + ba.astype(jnp.float32)
                sc = sc + bb.astype(jnp.float32)
                ba = jnp.zeros((_NB, 128), jnp.bfloat16)
                bb = jnp.zeros((_NB, 128), jnp.bfloat16)
        return sa, sc

    sp, st = accum2(x_ref, t_ref)
    sp = sp.reshape(1, _NB, 128)
    st = st.reshape(1, _NB, 128)

    @pl.when(j == 0)
    def _():
        ph_ref[...] = sp
        th_ref[...] = st

    @pl.when(j > 0)
    def _():
        ph_ref[...] += sp
        th_ref[...] += st


def _finalize_body(ph_ref, th_ref, out_ref):
    hp = jnp.sum(ph_ref[...], axis=-1)
    ht = jnp.sum(th_ref[...], axis=-1)
    np_sum = jnp.sum(hp, axis=-1, keepdims=True) + 1e-7
    nt_sum = jnp.sum(ht, axis=-1, keepdims=True) + 1e-7
    dn = hp / np_sum - ht / nt_sum
    row = jax.lax.broadcasted_iota(jnp.int32, (_NB, _NB), 0)
    col = jax.lax.broadcasted_iota(jnp.int32, (_NB, _NB), 1)
    tri = (row <= col).astype(jnp.float32)
    cum = jnp.dot(dn, tri, preferred_element_type=jnp.float32)
    a = jnp.sum(jnp.abs(cum), axis=-1, keepdims=True)
    tot = jnp.sum(a, axis=0, keepdims=True)
    out_ref[...] = tot * np.float32(1.0 / (dn.shape[0] * _NB))


@jax.jit
def kernel(pred, target):
    B, C, H, W = pred.shape
    bc = B * C
    hw = H * W
    rows128 = hw // 128
    rows = 576
    k_steps = rows128 // rows

    xp = pred.reshape(bc, rows128, 128)
    xt = target.reshape(bc, rows128, 128)

    ph, th = pl.pallas_call(
        functools.partial(_hist_body, rows=rows),
        grid=(bc, k_steps),
        in_specs=[
            pl.BlockSpec((1, rows, 128), lambda i, j: (i, j, 0)),
            pl.BlockSpec((1, rows, 128), lambda i, j: (i, j, 0)),
        ],
        out_specs=[
            pl.BlockSpec((1, _NB, 128), lambda i, j: (i, 0, 0)),
            pl.BlockSpec((1, _NB, 128), lambda i, j: (i, 0, 0)),
        ],
        out_shape=[
            jax.ShapeDtypeStruct((bc, _NB, 128), jnp.float32),
            jax.ShapeDtypeStruct((bc, _NB, 128), jnp.float32),
        ],
        compiler_params=pltpu.CompilerParams(
            dimension_semantics=("parallel", "arbitrary"),
        ),
    )(xp, xt)

    out = pl.pallas_call(
        _finalize_body,
        out_shape=jax.ShapeDtypeStruct((1, 1), jnp.float32),
    )(ph, th)

    return out[0, 0]


# speedup vs baseline: 1.0015x; 1.0015x over previous
import functools

import jax
import jax.numpy as jnp
import numpy as np
from jax.experimental import pallas as pl
from jax.experimental.pallas import tpu as pltpu

_NB = 64
_LOG2E = 1.4426950408889634
_BW = 1.0 / _NB
_DENOM = 2.0 * _BW * _BW + 1e-7
_SCALE = np.float32(np.sqrt(_LOG2E / _DENOM))
_CSTEP = np.float32(1.0 / (_NB - 1))


def _hist_body(x_ref, t_ref, ph_ref, th_ref, *, rows):
    j = pl.program_id(1)

    centers = (jax.lax.broadcasted_iota(jnp.int32, (_NB, 128), 0)
               .astype(jnp.float32) * np.float32(_CSTEP * _SCALE))

    def accum2(ref_a, ref_b):
        sa = jnp.zeros((_NB, 128), jnp.float32)
        sc = jnp.zeros((_NB, 128), jnp.float32)
        ba = jnp.zeros((_NB, 128), jnp.bfloat16)
        bb = jnp.zeros((_NB, 128), jnp.bfloat16)
        for k in range(rows):
            xa = ref_a[0, k : k + 1, :] * _SCALE
            xb = ref_b[0, k : k + 1, :] * _SCALE
            da = (xa - centers).astype(jnp.bfloat16)
            db = (xb - centers).astype(jnp.bfloat16)
            ba = ba + jnp.exp2(-(da * da))
            bb = bb + jnp.exp2(-(db * db))
            if (k + 1) % 32 == 0:
                sa = sa + ba.astype(jnp.float32)
                sc = sc + bb.astype(jnp.float32)
                ba = jnp.zeros((_NB, 128), jnp.bfloat16)
                bb = jnp.zeros((_NB, 128), jnp.bfloat16)
        return sa, sc

    sp, st = accum2(x_ref, t_ref)
    sp = sp.reshape(1, _NB, 128)
    st = st.reshape(1, _NB, 128)

    @pl.when(j == 0)
    def _():
        ph_ref[...] = sp
        th_ref[...] = st

    @pl.when(j > 0)
    def _():
        ph_ref[...] += sp
        th_ref[...] += st


def _finalize_body(ph_ref, th_ref, out_ref):
    hp = jnp.sum(ph_ref[...], axis=-1)
    ht = jnp.sum(th_ref[...], axis=-1)
    np_sum = jnp.sum(hp, axis=-1, keepdims=True) + 1e-7
    nt_sum = jnp.sum(ht, axis=-1, keepdims=True) + 1e-7
    dn = hp / np_sum - ht / nt_sum
    row = jax.lax.broadcasted_iota(jnp.int32, (_NB, _NB), 0)
    col = jax.lax.broadcasted_iota(jnp.int32, (_NB, _NB), 1)
    tri = (row <= col).astype(jnp.float32)
    cum = jnp.dot(dn, tri, preferred_element_type=jnp.float32)
    a = jnp.sum(jnp.abs(cum), axis=-1, keepdims=True)
    tot = jnp.sum(a, axis=0, keepdims=True)
    out_ref[...] = tot * np.float32(1.0 / (dn.shape[0] * _NB))


@jax.jit
def kernel(pred, target):
    B, C, H, W = pred.shape
    bc = B * C
    hw = H * W
    rows128 = hw // 128
    rows = 576
    k_steps = rows128 // rows

    xp = pred.reshape(bc, rows128, 128)
    xt = target.reshape(bc, rows128, 128)

    ph, th = pl.pallas_call(
        functools.partial(_hist_body, rows=rows),
        grid=(bc, k_steps),
        in_specs=[
            pl.BlockSpec((1, rows, 128), lambda i, j: (i, j, 0)),
            pl.BlockSpec((1, rows, 128), lambda i, j: (i, j, 0)),
        ],
        out_specs=[
            pl.BlockSpec((1, _NB, 128), lambda i, j: (i, 0, 0)),
            pl.BlockSpec((1, _NB, 128), lambda i, j: (i, 0, 0)),
        ],
        out_shape=[
            jax.ShapeDtypeStruct((bc, _NB, 128), jnp.float32),
            jax.ShapeDtypeStruct((bc, _NB, 128), jnp.float32),
        ],
        compiler_params=pltpu.CompilerParams(
            dimension_semantics=("parallel", "arbitrary"),
        ),
    )(xp, xt)

    out = pl.pallas_call(
        _finalize_body,
        out_shape=jax.ShapeDtypeStruct((1, 1), jnp.float32),
    )(ph, th)

    return out[0, 0]
